# Initial kernel scaffold; baseline (speedup 1.0000x reference)
import jax
import jax.numpy as jnp
from jax.experimental import pallas as pl


def kernel(item_ids, table):
    raise NotImplementedError("write your pallas kernel here")


# speedup vs baseline: 98.6920x; 98.6920x over previous
import functools

import jax
import jax.numpy as jnp
from jax import lax
from jax.experimental import pallas as pl
from jax.experimental.pallas import tpu as pltpu
from jax.experimental.pallas import tpu_sc as plsc

NC = 2
NS = 16
NW = NC * NS

GATHER_B = 128
G = 16
CHUNK = GATHER_B * G


def _gather_body(idx_hbm, table_hbm, out_hbm, idx_v, out_v, lsem, gsem, osem,
                 *, n_per_w):
    wid = lax.axis_index("s") * NC + lax.axis_index("c")
    base = wid * n_per_w
    n_chunks = n_per_w // CHUNK

    def step(i, _):
        off = base + i * CHUNK
        cp = pltpu.async_copy(idx_hbm.at[pl.ds(off, CHUNK)], idx_v, lsem)
        cp.wait()
        cps = [
            pltpu.async_copy(
                table_hbm.at[idx_v.at[pl.ds(j * GATHER_B, GATHER_B)]],
                out_v.at[pl.ds(j * GATHER_B, GATHER_B)],
                gsem,
            )
            for j in range(G)
        ]
        for cp in cps:
            cp.wait()
        pltpu.async_copy(out_v, out_hbm.at[pl.ds(off, CHUNK)], osem).wait()
        return ()

    lax.fori_loop(0, n_chunks, step, (), unroll=False)


def kernel(item_ids, table):
    n_total = item_ids.shape[0] * item_ids.shape[1]
    flat_idx = item_ids.reshape(n_total).astype(jnp.int32)
    flat_table = table.reshape(table.shape[0])
    n_per_w = n_total // NW

    mesh = plsc.VectorSubcoreMesh(core_axis_name="c", subcore_axis_name="s")
    flat_out = pl.kernel(
        functools.partial(_gather_body, n_per_w=n_per_w),
        out_type=jax.ShapeDtypeStruct((n_total,), jnp.float32),
        mesh=mesh,
        scratch_types=[
            pltpu.VMEM((CHUNK,), jnp.int32),
            pltpu.VMEM((CHUNK,), jnp.float32),
            pltpu.SemaphoreType.DMA,
            pltpu.SemaphoreType.DMA,
            pltpu.SemaphoreType.DMA,
        ],
    )(flat_idx, flat_table)
    return flat_out.reshape(item_ids.shape[0], item_ids.shape[1], 1)


# speedup vs baseline: 98.7918x; 1.0010x over previous
import functools

import jax
import jax.numpy as jnp
from jax import lax
from jax.experimental import pallas as pl
from jax.experimental.pallas import tpu as pltpu
from jax.experimental.pallas import tpu_sc as plsc

NC = 2
NS = 16
NW = NC * NS

CHUNK = 2048


def _gather_body(idx_hbm, table_hbm, out_hbm, idx_v, out_v, lsem, gsem, osem,
                 *, rows_per_w):
    wid = lax.axis_index("s") * NC + lax.axis_index("c")
    base = wid * rows_per_w
    n_blocks = rows_per_w // CHUNK

    def step(i, _):
        off = base + i * CHUNK
        pltpu.async_copy(idx_hbm.at[pl.ds(off, CHUNK)], idx_v, lsem).wait()
        pltpu.async_copy(table_hbm.at[idx_v], out_v, gsem).wait()
        pltpu.async_copy(out_v, out_hbm.at[pl.ds(off, CHUNK)], osem).wait()
        return ()

    lax.fori_loop(0, n_blocks, step, (), unroll=False)


def kernel(item_ids, table):
    n_total = item_ids.shape[0] * item_ids.shape[1]
    flat_idx = item_ids.reshape(n_total).astype(jnp.int32)
    flat_table = table.reshape(table.shape[0])
    rows_per_w = n_total // NW

    mesh = plsc.VectorSubcoreMesh(core_axis_name="c", subcore_axis_name="s")
    flat_out = pl.kernel(
        functools.partial(_gather_body, rows_per_w=rows_per_w),
        out_type=jax.ShapeDtypeStruct((n_total,), jnp.float32),
        mesh=mesh,
        scratch_types=[
            pltpu.VMEM((CHUNK,), jnp.int32),
            pltpu.VMEM((CHUNK,), jnp.float32),
            pltpu.SemaphoreType.DMA,
            pltpu.SemaphoreType.DMA,
            pltpu.SemaphoreType.DMA,
        ],
    )(flat_idx, flat_table)
    return flat_out.reshape(item_ids.shape[0], item_ids.shape[1], 1)


# speedup vs baseline: 120.5371x; 1.2201x over previous
import functools

import jax
import jax.numpy as jnp
from jax import lax
from jax.experimental import pallas as pl
from jax.experimental.pallas import tpu as pltpu
from jax.experimental.pallas import tpu_sc as plsc

NC = 2
NS = 16
NW = NC * NS

CHUNK = 12800


def _gather_body(idx_hbm, table_hbm, out_hbm, idx_v, out_v, lsem, gsem, osem,
                 *, rows_per_w):
    wid = lax.axis_index("s") * NC + lax.axis_index("c")
    base = wid * rows_per_w
    n_blocks = rows_per_w // CHUNK

    def step(i, _):
        off = base + i * CHUNK
        pltpu.async_copy(idx_hbm.at[pl.ds(off, CHUNK)], idx_v, lsem).wait()
        pltpu.async_copy(table_hbm.at[idx_v], out_v, gsem).wait()
        pltpu.async_copy(out_v, out_hbm.at[pl.ds(off, CHUNK)], osem).wait()
        return ()

    lax.fori_loop(0, n_blocks, step, (), unroll=False)


def kernel(item_ids, table):
    n_total = item_ids.shape[0] * item_ids.shape[1]
    flat_idx = item_ids.reshape(n_total).astype(jnp.int32)
    flat_table = table.reshape(table.shape[0])
    rows_per_w = n_total // NW

    mesh = plsc.VectorSubcoreMesh(core_axis_name="c", subcore_axis_name="s")
    flat_out = pl.kernel(
        functools.partial(_gather_body, rows_per_w=rows_per_w),
        out_type=jax.ShapeDtypeStruct((n_total,), jnp.float32),
        mesh=mesh,
        scratch_types=[
            pltpu.VMEM((CHUNK,), jnp.int32),
            pltpu.VMEM((CHUNK,), jnp.float32),
            pltpu.SemaphoreType.DMA,
            pltpu.SemaphoreType.DMA,
            pltpu.SemaphoreType.DMA,
        ],
    )(flat_idx, flat_table)
    return flat_out.reshape(item_ids.shape[0], item_ids.shape[1], 1)


# speedup vs baseline: 176.8419x; 1.4671x over previous
import functools

import jax
import jax.numpy as jnp
from jax import lax
from jax.experimental import pallas as pl
from jax.experimental.pallas import tpu as pltpu
from jax.experimental.pallas import tpu_sc as plsc

NC = 2
NS = 16
NW = NC * NS

CHUNK = 12800
TBL_SLICE = 62720
TBL_BOUNCE = 15680
TBL_PAD = TBL_SLICE * NS


def _gather_body(idx_hbm, table_hbm, out_hbm, idx_v, out_v, tbl_sh, tbl_b,
                 lsem, gsem, osem, tsem, *, rows_per_w):
    cid = lax.axis_index("c")
    sid = lax.axis_index("s")
    wid = sid * NC + cid
    base = wid * rows_per_w
    n_blocks = rows_per_w // CHUNK

    for r in range(TBL_SLICE // TBL_BOUNCE):
        t_off = sid * TBL_SLICE + r * TBL_BOUNCE
        pltpu.async_copy(
            table_hbm.at[pl.ds(t_off, TBL_BOUNCE)], tbl_b, tsem).wait()
        pltpu.async_copy(
            tbl_b, tbl_sh.at[pl.ds(t_off, TBL_BOUNCE)], tsem).wait()
    plsc.subcore_barrier()

    def step(i, _):
        off = base + i * CHUNK
        pltpu.async_copy(idx_hbm.at[pl.ds(off, CHUNK)], idx_v, lsem).wait()
        pltpu.async_copy(tbl_sh.at[idx_v], out_v, gsem).wait()
        pltpu.async_copy(out_v, out_hbm.at[pl.ds(off, CHUNK)], osem).wait()
        return ()

    lax.fori_loop(0, n_blocks, step, (), unroll=False)


def kernel(item_ids, table):
    n_total = item_ids.shape[0] * item_ids.shape[1]
    flat_idx = item_ids.reshape(n_total).astype(jnp.int32)
    flat_table = table.reshape(table.shape[0])
    flat_table = jnp.pad(flat_table, (0, TBL_PAD - flat_table.shape[0]))
    rows_per_w = n_total // NW

    mesh = plsc.VectorSubcoreMesh(core_axis_name="c", subcore_axis_name="s")
    flat_out = pl.kernel(
        functools.partial(_gather_body, rows_per_w=rows_per_w),
        out_type=jax.ShapeDtypeStruct((n_total,), jnp.float32),
        mesh=mesh,
        scratch_types=[
            pltpu.VMEM((CHUNK,), jnp.int32),
            pltpu.VMEM((CHUNK,), jnp.float32),
            pltpu.MemorySpace.VMEM_SHARED((TBL_PAD,), jnp.float32),
            pltpu.VMEM((TBL_BOUNCE,), jnp.float32),
            pltpu.SemaphoreType.DMA,
            pltpu.SemaphoreType.DMA,
            pltpu.SemaphoreType.DMA,
            pltpu.SemaphoreType.DMA,
        ],
    )(flat_idx, flat_table)
    return flat_out.reshape(item_ids.shape[0], item_ids.shape[1], 1)


# speedup vs baseline: 189.2708x; 1.0703x over previous
import functools

import jax
import jax.numpy as jnp
from jax import lax
from jax.experimental import pallas as pl
from jax.experimental.pallas import tpu as pltpu
from jax.experimental.pallas import tpu_sc as plsc

NC = 2
NS = 16
NW = NC * NS

CHUNK = 12800
NB = 2
TBL_SLICE = 62720
TBL_BOUNCE = 15680
TBL_PAD = TBL_SLICE * NS


def _gather_body(idx_hbm, table_hbm, out_hbm,
                 idx0, idx1, out0, out1, tbl_sh, tbl_b,
                 l0, l1, g0, g1, o0, o1, tsem, *, rows_per_w):
    cid = lax.axis_index("c")
    sid = lax.axis_index("s")
    wid = sid * NC + cid
    base = wid * rows_per_w
    n = rows_per_w // CHUNK

    idx_v = [idx0, idx1]
    out_v = [out0, out1]
    lsem = [l0, l1]
    gsem = [g0, g1]
    osem = [o0, o1]

    cp_load, cp_gather, cp_store = {}, {}, {}

    def fire_load(c):
        cp_load[c] = pltpu.async_copy(
            idx_hbm.at[pl.ds(base + c * CHUNK, CHUNK)], idx_v[c % NB],
            lsem[c % NB])

    def fire_gather(c):
        cp_gather[c] = pltpu.async_copy(
            tbl_sh.at[idx_v[c % NB]], out_v[c % NB], gsem[c % NB])

    def fire_store(c):
        cp_store[c] = pltpu.async_copy(
            out_v[c % NB], out_hbm.at[pl.ds(base + c * CHUNK, CHUNK)],
            osem[c % NB])

    fire_load(0)
    fire_load(1)

    for r in range(TBL_SLICE // TBL_BOUNCE):
        t_off = sid * TBL_SLICE + r * TBL_BOUNCE
        pltpu.async_copy(
            table_hbm.at[pl.ds(t_off, TBL_BOUNCE)], tbl_b, tsem).wait()
        pltpu.async_copy(
            tbl_b, tbl_sh.at[pl.ds(t_off, TBL_BOUNCE)], tsem).wait()
    plsc.subcore_barrier()

    cp_load[0].wait()
    fire_gather(0)
    for c in range(n):
        if c + 1 < n:
            cp_load[c + 1].wait()
            if c - 1 >= 0:
                cp_store[c - 1].wait()
            fire_gather(c + 1)
        cp_gather[c].wait()
        fire_store(c)
        if c + NB < n:
            fire_load(c + NB)
    cp_store[n - 2].wait()
    cp_store[n - 1].wait()


def kernel(item_ids, table):
    n_total = item_ids.shape[0] * item_ids.shape[1]
    flat_idx = item_ids.reshape(n_total).astype(jnp.int32)
    flat_table = table.reshape(table.shape[0])
    flat_table = jnp.pad(flat_table, (0, TBL_PAD - flat_table.shape[0]))
    rows_per_w = n_total // NW

    mesh = plsc.VectorSubcoreMesh(core_axis_name="c", subcore_axis_name="s")
    flat_out = pl.kernel(
        functools.partial(_gather_body, rows_per_w=rows_per_w),
        out_type=jax.ShapeDtypeStruct((n_total,), jnp.float32),
        mesh=mesh,
        scratch_types=[
            pltpu.VMEM((CHUNK,), jnp.int32),
            pltpu.VMEM((CHUNK,), jnp.int32),
            pltpu.VMEM((CHUNK,), jnp.float32),
            pltpu.VMEM((CHUNK,), jnp.float32),
            pltpu.MemorySpace.VMEM_SHARED((TBL_PAD,), jnp.float32),
            pltpu.VMEM((TBL_BOUNCE,), jnp.float32),
            pltpu.SemaphoreType.DMA,
            pltpu.SemaphoreType.DMA,
            pltpu.SemaphoreType.DMA,
            pltpu.SemaphoreType.DMA,
            pltpu.SemaphoreType.DMA,
            pltpu.SemaphoreType.DMA,
            pltpu.SemaphoreType.DMA,
        ],
    )(flat_idx, flat_table)
    return flat_out.reshape(item_ids.shape[0], item_ids.shape[1], 1)


# speedup vs baseline: 307.9285x; 1.6269x over previous
import functools

import jax
import jax.numpy as jnp
from jax import lax
from jax.experimental import pallas as pl
from jax.experimental.pallas import tpu as pltpu
from jax.experimental.pallas import tpu_sc as plsc

NC = 2
NS = 16
NW = NC * NS

CHUNK = 12800
NB = 2
TBL_SLICE = 62720
TBL_BOUNCE = 15680
TBL_PAD = TBL_SLICE * NS


def _gather_body(idx_hbm, table_hbm, out_hbm,
                 idx0, idx1, out0, out1, tbl_sh, tbl_b,
                 l0, l1, g0, g1, o0, o1, tsem, *, rows_per_w):
    cid = lax.axis_index("c")
    sid = lax.axis_index("s")
    wid = sid * NC + cid
    base = wid * rows_per_w
    n = rows_per_w // CHUNK

    idx_v = [idx0, idx1]
    out_v = [out0, out1]
    lsem = [l0, l1]
    gsem = [g0, g1]
    osem = [o0, o1]

    cp_load, cp_gather, cp_store = {}, {}, {}

    def fire_load(c):
        cp_load[c] = pltpu.async_copy(
            idx_hbm.at[pl.ds(base + c * CHUNK, CHUNK)], idx_v[c % NB],
            lsem[c % NB])

    def fire_gather(c):
        cp_gather[c] = pltpu.async_copy(
            tbl_sh.at[idx_v[c % NB]], out_v[c % NB], gsem[c % NB])

    def fire_store(c):
        cp_store[c] = pltpu.async_copy(
            out_v[c % NB], out_hbm.at[pl.ds(base + c * CHUNK, CHUNK)],
            osem[c % NB])

    fire_load(0)
    fire_load(1)

    for r in range(TBL_SLICE // TBL_BOUNCE):
        t_off = sid * TBL_SLICE + r * TBL_BOUNCE
        pltpu.async_copy(
            table_hbm.at[pl.ds(t_off, TBL_BOUNCE)], tbl_b, tsem).wait()
        pltpu.async_copy(
            tbl_b, tbl_sh.at[pl.ds(t_off, TBL_BOUNCE)], tsem).wait()
    plsc.subcore_barrier()

    cp_load[0].wait()
    fire_gather(0)
    for c in range(n):
        if c + 1 < n:
            cp_load[c + 1].wait()
            if c - 1 >= 0:
                cp_store[c - 1].wait()
            fire_gather(c + 1)
        cp_gather[c].wait()
        fire_store(c)
        if c + NB < n:
            fire_load(c + NB)
    cp_store[n - 2].wait()
    cp_store[n - 1].wait()


def kernel(item_ids, table):
    n_total = item_ids.shape[0] * item_ids.shape[1]
    flat_idx = item_ids.T.reshape(n_total)
    flat_table = table.reshape(table.shape[0])
    flat_table = jnp.pad(flat_table, (0, TBL_PAD - flat_table.shape[0]))
    rows_per_w = n_total // NW

    mesh = plsc.VectorSubcoreMesh(core_axis_name="c", subcore_axis_name="s")
    flat_out = pl.kernel(
        functools.partial(_gather_body, rows_per_w=rows_per_w),
        out_type=jax.ShapeDtypeStruct((n_total,), jnp.float32),
        mesh=mesh,
        scratch_types=[
            pltpu.VMEM((CHUNK,), jnp.int32),
            pltpu.VMEM((CHUNK,), jnp.int32),
            pltpu.VMEM((CHUNK,), jnp.float32),
            pltpu.VMEM((CHUNK,), jnp.float32),
            pltpu.MemorySpace.VMEM_SHARED((TBL_PAD,), jnp.float32),
            pltpu.VMEM((TBL_BOUNCE,), jnp.float32),
            pltpu.SemaphoreType.DMA,
            pltpu.SemaphoreType.DMA,
            pltpu.SemaphoreType.DMA,
            pltpu.SemaphoreType.DMA,
            pltpu.SemaphoreType.DMA,
            pltpu.SemaphoreType.DMA,
            pltpu.SemaphoreType.DMA,
        ],
    )(flat_idx, flat_table)
    return flat_out.reshape(
        item_ids.shape[1], item_ids.shape[0], 1).transpose(1, 0, 2)


# speedup vs baseline: 412.4408x; 1.3394x over previous
import functools

import jax
import jax.numpy as jnp
from jax import lax
from jax.experimental import pallas as pl
from jax.experimental.pallas import tpu as pltpu
from jax.experimental.pallas import tpu_sc as plsc

NC = 2
NS = 16
NW = NC * NS

CHUNK = 12800
NB = 2
TBL_SLICE = 62592
TBL_BOUNCE = 15648
TBL_PAD = TBL_SLICE * NS


def _gather_body(idx_hbm, table_hbm, out_hbm,
                 idx0, idx1, out0, out1, tbl_sh, tbl_b,
                 l0, l1, g0, g1, o0, o1, tsem, *, rows_per_w):
    cid = lax.axis_index("c")
    sid = lax.axis_index("s")
    wid = sid * NC + cid
    base = wid * rows_per_w
    n = rows_per_w // CHUNK

    idx_v = [idx0, idx1]
    out_v = [out0, out1]
    lsem = [l0, l1]
    gsem = [g0, g1]
    osem = [o0, o1]

    cp_load, cp_gather, cp_store = {}, {}, {}

    def fire_load(c):
        cp_load[c] = pltpu.async_copy(
            idx_hbm.at[pl.ds(base + c * CHUNK, CHUNK)], idx_v[c % NB],
            lsem[c % NB])

    def fire_gather(c):
        cp_gather[c] = pltpu.async_copy(
            tbl_sh.at[idx_v[c % NB]], out_v[c % NB], gsem[c % NB])

    def fire_store(c):
        cp_store[c] = pltpu.async_copy(
            out_v[c % NB], out_hbm.at[pl.ds(base + c * CHUNK, CHUNK)],
            osem[c % NB])

    fire_load(0)
    fire_load(1)

    for r in range(TBL_SLICE // TBL_BOUNCE):
        t_off = sid * TBL_SLICE + r * TBL_BOUNCE
        pltpu.async_copy(
            table_hbm.at[pl.ds(t_off, TBL_BOUNCE)], tbl_b, tsem).wait()
        pltpu.async_copy(
            tbl_b, tbl_sh.at[pl.ds(t_off, TBL_BOUNCE)], tsem).wait()
    plsc.subcore_barrier()

    cp_load[0].wait()
    fire_gather(0)
    for c in range(n):
        if c + 1 < n:
            cp_load[c + 1].wait()
            if c - 1 >= 0:
                cp_store[c - 1].wait()
            fire_gather(c + 1)
        cp_gather[c].wait()
        fire_store(c)
        if c + NB < n:
            fire_load(c + NB)
    cp_store[n - 2].wait()
    cp_store[n - 1].wait()


def kernel(item_ids, table):
    n_total = item_ids.shape[0] * item_ids.shape[1]
    flat_idx = item_ids.T.reshape(n_total)
    flat_table = jnp.pad(
        table, ((0, TBL_PAD - table.shape[0]), (0, 0))).reshape(TBL_PAD)
    rows_per_w = n_total // NW

    mesh = plsc.VectorSubcoreMesh(core_axis_name="c", subcore_axis_name="s")
    flat_out = pl.kernel(
        functools.partial(_gather_body, rows_per_w=rows_per_w),
        out_type=jax.ShapeDtypeStruct((n_total,), jnp.float32),
        mesh=mesh,
        scratch_types=[
            pltpu.VMEM((CHUNK,), jnp.int32),
            pltpu.VMEM((CHUNK,), jnp.int32),
            pltpu.VMEM((CHUNK,), jnp.float32),
            pltpu.VMEM((CHUNK,), jnp.float32),
            pltpu.MemorySpace.VMEM_SHARED((TBL_PAD,), jnp.float32),
            pltpu.VMEM((TBL_BOUNCE,), jnp.float32),
            pltpu.SemaphoreType.DMA,
            pltpu.SemaphoreType.DMA,
            pltpu.SemaphoreType.DMA,
            pltpu.SemaphoreType.DMA,
            pltpu.SemaphoreType.DMA,
            pltpu.SemaphoreType.DMA,
            pltpu.SemaphoreType.DMA,
        ],
    )(flat_idx, flat_table)
    return flat_out.reshape(
        item_ids.shape[1], item_ids.shape[0], 1).transpose(1, 0, 2)
